# baseline (speedup 1.0000x reference)
import functools

import jax
import jax.numpy as jnp
from jax import lax
from jax.experimental import pallas as pl
from jax.experimental.pallas import tpu as pltpu
from jax.experimental.pallas import tpu_sc as plsc

N_ENTITIES = 1000000
N_RELATIONS = 1000
C = 64
T = 16384

NC = 2
NS = 16
L = 16
NW = NC * NS
TPW = T // NW
CH = 128
NCHUNK = TPW // CH

_mesh = plsc.VectorSubcoreMesh(core_axis_name="c", subcore_axis_name="s")


@functools.partial(
    pl.kernel,
    mesh=_mesh,
    compiler_params=pltpu.CompilerParams(
        needs_layout_passes=False, use_tc_tiling_on_sc=False
    ),
    out_type=jax.ShapeDtypeStruct((T,), jnp.float32),
    scratch_types=[
        pltpu.VMEM((NCHUNK, CH), jnp.int32),
        pltpu.VMEM((NCHUNK, CH), jnp.int32),
        pltpu.VMEM((NCHUNK, CH), jnp.int32),
        pltpu.VMEM((TPW, C), jnp.float32),
        pltpu.VMEM((TPW, C), jnp.float32),
        pltpu.VMEM((TPW, C), jnp.float32),
        pltpu.VMEM((TPW,), jnp.float32),
        pltpu.SemaphoreType.DMA,
    ],
)
def _distmult_sc(ent_hbm, rel_hbm, s_hbm, r_hbm, o_hbm, out_hbm,
                 sidx, ridx, oidx, es_v, wr_v, eo_v, out_v, sem):
    wid = lax.axis_index("s") * NC + lax.axis_index("c")
    base = wid * TPW

    pltpu.sync_copy(s_hbm.at[wid], sidx)
    pltpu.sync_copy(r_hbm.at[wid], ridx)
    pltpu.sync_copy(o_hbm.at[wid], oidx)

    copies = []
    for j in range(NCHUNK):
        dst = pl.ds(j * CH, CH)
        copies.append(pltpu.async_copy(ent_hbm.at[sidx.at[j]], es_v.at[dst, :], sem))
        copies.append(pltpu.async_copy(rel_hbm.at[ridx.at[j]], wr_v.at[dst, :], sem))
        copies.append(pltpu.async_copy(ent_hbm.at[oidx.at[j]], eo_v.at[dst, :], sem))
    for cp in copies:
        cp.wait()

    lanes = lax.iota(jnp.int32, L)

    def body(g, carry):
        rows = g * L + lanes
        acc = jnp.zeros((L,), jnp.float32)
        for c in range(C):
            col = jnp.full((L,), c, jnp.int32)
            a = plsc.load_gather(es_v, [rows, col])
            b = plsc.load_gather(wr_v, [rows, col])
            d = plsc.load_gather(eo_v, [rows, col])
            acc = acc + a * b * d
        out_v[pl.ds(g * L, L)] = acc
        return carry

    lax.fori_loop(0, TPW // L, body, 0)

    pltpu.sync_copy(out_v, out_hbm.at[pl.ds(base, TPW)])


def kernel(initializations, rel_weights, sro_triples):
    s = sro_triples[0].reshape(NW, NCHUNK, CH)
    r = sro_triples[1].reshape(NW, NCHUNK, CH)
    o = sro_triples[2].reshape(NW, NCHUNK, CH)
    return _distmult_sc(initializations, rel_weights, s, r, o)


# speedup vs baseline: 1.0023x; 1.0023x over previous
import functools

import jax
import jax.numpy as jnp
from jax import lax
from jax.experimental import pallas as pl
from jax.experimental.pallas import tpu as pltpu
from jax.experimental.pallas import tpu_sc as plsc

N_ENTITIES = 1000000
N_RELATIONS = 1000
C = 64
T = 16384

NC = 2
NS = 16
L = 16
NW = NC * NS
TPW = T // NW
CH = 128
NCHUNK = TPW // CH

_mesh = plsc.VectorSubcoreMesh(core_axis_name="c", subcore_axis_name="s")


@functools.partial(
    pl.kernel,
    mesh=_mesh,
    compiler_params=pltpu.CompilerParams(
        needs_layout_passes=False, use_tc_tiling_on_sc=False
    ),
    out_type=jax.ShapeDtypeStruct((T,), jnp.float32),
    scratch_types=[
        pltpu.VMEM((NCHUNK, CH), jnp.int32),
        pltpu.VMEM((NCHUNK, CH), jnp.int32),
        pltpu.VMEM((NCHUNK, CH), jnp.int32),
        pltpu.VMEM((TPW, C), jnp.float32),
        pltpu.VMEM((TPW, C), jnp.float32),
        pltpu.VMEM((TPW, C), jnp.float32),
        pltpu.VMEM((TPW,), jnp.float32),
        pltpu.SemaphoreType.DMA,
    ],
)
def _distmult_sc(ent_hbm, rel_hbm, s_hbm, r_hbm, o_hbm, out_hbm,
                 sidx, ridx, oidx, es_v, wr_v, eo_v, out_v, sem):
    wid = lax.axis_index("s") * NC + lax.axis_index("c")
    base = wid * TPW

    pltpu.sync_copy(s_hbm.at[wid], sidx)
    pltpu.sync_copy(r_hbm.at[wid], ridx)
    pltpu.sync_copy(o_hbm.at[wid], oidx)

    copies = []
    for j in range(NCHUNK):
        dst = pl.ds(j * CH, CH)
        copies.append(pltpu.async_copy(ent_hbm.at[sidx.at[j]], es_v.at[dst, :], sem))
        copies.append(pltpu.async_copy(rel_hbm.at[ridx.at[j]], wr_v.at[dst, :], sem))
        copies.append(pltpu.async_copy(ent_hbm.at[oidx.at[j]], eo_v.at[dst, :], sem))
    for cp in copies:
        cp.wait()

    lanes = lax.iota(jnp.int32, L)

    def body(g, carry):
        rows = g * L + lanes
        acc = jnp.zeros((L,), jnp.float32)
        for c in range(C):
            col = jnp.full((L,), c, jnp.int32)
            a = plsc.load_gather(es_v, [rows, col])
            b = plsc.load_gather(wr_v, [rows, col])
            d = plsc.load_gather(eo_v, [rows, col])
            acc = acc + a * b * d
        out_v[pl.ds(g * L, L)] = acc
        return carry

    lax.fori_loop(0, TPW // L, body, 0)

    pltpu.sync_copy(out_v, out_hbm.at[pl.ds(base, TPW)])


def kernel(initializations, rel_weights, sro_triples):
    ent = jax.lax.optimization_barrier(initializations.reshape(-1))
    ent = ent.reshape(N_ENTITIES, C)
    rel = jax.lax.optimization_barrier(rel_weights.reshape(-1))
    rel = rel.reshape(N_RELATIONS, C)
    s = sro_triples[0].reshape(NW, NCHUNK, CH)
    r = sro_triples[1].reshape(NW, NCHUNK, CH)
    o = sro_triples[2].reshape(NW, NCHUNK, CH)
    return _distmult_sc(ent, rel, s, r, o)


# speedup vs baseline: 1.0560x; 1.0536x over previous
import functools

import jax
import jax.numpy as jnp
from jax import lax
from jax.experimental import pallas as pl
from jax.experimental.pallas import tpu as pltpu
from jax.experimental.pallas import tpu_sc as plsc

N_ENTITIES = 1000000
N_RELATIONS = 1000
C = 64
T = 16384

NC = 2
NS = 16
L = 16
NW = NC * NS
TPW = T // NW
CH = 128
NCHUNK = TPW // CH
PR = 2 * C

_mesh = plsc.VectorSubcoreMesh(core_axis_name="c", subcore_axis_name="s")


@functools.partial(
    pl.kernel,
    mesh=_mesh,
    compiler_params=pltpu.CompilerParams(needs_layout_passes=False),
    out_type=jax.ShapeDtypeStruct((T,), jnp.float32),
    scratch_types=[
        pltpu.VMEM((NCHUNK, CH), jnp.int32),
        pltpu.VMEM((NCHUNK, CH), jnp.int32),
        pltpu.VMEM((NCHUNK, CH), jnp.int32),
        pltpu.VMEM((NCHUNK, CH), jnp.int32),
        pltpu.VMEM((NCHUNK, CH), jnp.int32),
        pltpu.VMEM((NCHUNK, CH), jnp.int32),
        pltpu.VMEM((NCHUNK, CH), jnp.int32),
        pltpu.VMEM((NCHUNK, CH), jnp.int32),
        pltpu.VMEM((NCHUNK, CH), jnp.int32),
        pltpu.VMEM((CH, PR), jnp.float32),
        pltpu.VMEM((CH, PR), jnp.float32),
        pltpu.VMEM((CH, PR), jnp.float32),
        pltpu.VMEM((TPW,), jnp.float32),
        pltpu.SemaphoreType.DMA,
    ],
)
def _distmult_sc(ent_hbm, rel_hbm, s_hbm, r_hbm, o_hbm, out_hbm,
                 sidx, ridx, oidx, srow, rrow, orow, shalf, rhalf, ohalf,
                 es_v, wr_v, eo_v, out_v, sem):
    wid = lax.axis_index("s") * NC + lax.axis_index("c")
    base = wid * TPW

    pltpu.sync_copy(s_hbm.at[wid], sidx)
    pltpu.sync_copy(r_hbm.at[wid], ridx)
    pltpu.sync_copy(o_hbm.at[wid], oidx)

    for raw, row, half in ((sidx, srow, shalf), (ridx, rrow, rhalf),
                           (oidx, orow, ohalf)):
        for j in range(NCHUNK):
            for k in range(CH // L):
                v = raw[j, pl.ds(k * L, L)]
                row[j, pl.ds(k * L, L)] = v >> 1
                half[j, pl.ds(k * L, L)] = (v & 1) * C

    def chunk_body(j, carry):
        cs = pltpu.async_copy(ent_hbm.at[srow.at[j]], es_v, sem)
        cr = pltpu.async_copy(rel_hbm.at[rrow.at[j]], wr_v, sem)
        co = pltpu.async_copy(ent_hbm.at[orow.at[j]], eo_v, sem)
        cs.wait()
        cr.wait()
        co.wait()

        lanes = lax.iota(jnp.int32, L)

        def group_body(g, carry2):
            hs_v = shalf[j, pl.ds(g * L, L)]
            hr_v = rhalf[j, pl.ds(g * L, L)]
            ho_v = ohalf[j, pl.ds(g * L, L)]
            sums = jnp.zeros((L,), jnp.float32)
            for tloc in range(L):
                t = g * L + tloc
                hs = hs_v[tloc]
                hr = hr_v[tloc]
                ho = ho_v[tloc]
                acc = jnp.zeros((L,), jnp.float32)
                for k in range(C // L):
                    a = es_v[t, pl.ds(hs + k * L, L)]
                    b = wr_v[t, pl.ds(hr + k * L, L)]
                    d = eo_v[t, pl.ds(ho + k * L, L)]
                    acc = acc + a * b * d
                sums = jnp.where(lanes == tloc, jnp.sum(acc), sums)
            out_v[pl.ds(j * CH + g * L, L)] = sums
            return carry2

        lax.fori_loop(0, CH // L, group_body, 0)
        return carry

    lax.fori_loop(0, NCHUNK, chunk_body, 0)

    pltpu.sync_copy(out_v, out_hbm.at[pl.ds(base, TPW)])


def kernel(initializations, rel_weights, sro_triples):
    ent2 = initializations.reshape(N_ENTITIES // 2, PR)
    rel2 = rel_weights.reshape(N_RELATIONS // 2, PR)
    s = sro_triples[0].reshape(NW, NCHUNK, CH)
    r = sro_triples[1].reshape(NW, NCHUNK, CH)
    o = sro_triples[2].reshape(NW, NCHUNK, CH)
    return _distmult_sc(ent2, rel2, s, r, o)


# speedup vs baseline: 1.7881x; 1.6933x over previous
import functools

import jax
import jax.numpy as jnp
from jax import lax
from jax.experimental import pallas as pl
from jax.experimental.pallas import tpu as pltpu
from jax.experimental.pallas import tpu_sc as plsc

N_ENTITIES = 1000000
N_RELATIONS = 1000
C = 64
T = 16384

NC = 2
NS = 16
L = 16
NW = NC * NS
TPW = T // NW
CH = 128
NCHUNK = TPW // CH

_mesh = plsc.VectorSubcoreMesh(core_axis_name="c", subcore_axis_name="s")


@functools.partial(
    pl.kernel,
    mesh=_mesh,
    compiler_params=pltpu.CompilerParams(needs_layout_passes=False),
    out_type=jax.ShapeDtypeStruct((T,), jnp.float32),
    scratch_types=[
        pltpu.VMEM((NCHUNK, CH), jnp.int32),
        pltpu.VMEM((NCHUNK, CH), jnp.int32),
        pltpu.VMEM((NCHUNK, CH), jnp.int32),
        pltpu.VMEM((2, CH, C), jnp.float32),
        pltpu.VMEM((2, CH, C), jnp.float32),
        pltpu.VMEM((2, CH, C), jnp.float32),
        pltpu.VMEM((TPW,), jnp.float32),
        pltpu.SemaphoreType.DMA,
        pltpu.SemaphoreType.DMA,
    ],
)
def _distmult_sc(ent_hbm, rel_hbm, s_hbm, r_hbm, o_hbm, out_hbm,
                 sidx, ridx, oidx, es_v, wr_v, eo_v, out_v, sem0, sem1):
    wid = lax.axis_index("s") * NC + lax.axis_index("c")
    base = wid * TPW

    pltpu.sync_copy(s_hbm.at[wid], sidx)
    pltpu.sync_copy(r_hbm.at[wid], ridx)
    pltpu.sync_copy(o_hbm.at[wid], oidx)

    sems = (sem0, sem1)

    def fire_chunk(j, slot, sem):
        def fire_group(g, carry):
            sv = sidx[j, pl.ds(g * L, L)]
            rv = ridx[j, pl.ds(g * L, L)]
            ov = oidx[j, pl.ds(g * L, L)]
            t0 = g * L
            for tloc in range(L):
                pltpu.async_copy(
                    ent_hbm.at[pl.ds(sv[tloc], 1), :],
                    es_v.at[slot, pl.ds(t0 + tloc, 1), :], sem)
                pltpu.async_copy(
                    rel_hbm.at[pl.ds(rv[tloc], 1), :],
                    wr_v.at[slot, pl.ds(t0 + tloc, 1), :], sem)
                pltpu.async_copy(
                    ent_hbm.at[pl.ds(ov[tloc], 1), :],
                    eo_v.at[slot, pl.ds(t0 + tloc, 1), :], sem)
            return carry

        lax.fori_loop(0, CH // L, fire_group, 0)

    def drain_chunk(slot, sem):
        pltpu.make_async_copy(
            ent_hbm.at[pl.ds(0, CH), :], es_v.at[slot], sem).wait()
        pltpu.make_async_copy(
            rel_hbm.at[pl.ds(0, CH), :], wr_v.at[slot], sem).wait()
        pltpu.make_async_copy(
            ent_hbm.at[pl.ds(0, CH), :], eo_v.at[slot], sem).wait()

    lanes = lax.iota(jnp.int32, L)

    def compute_chunk(j, slot):
        def group_body(g, carry2):
            sums = jnp.zeros((L,), jnp.float32)
            for tloc in range(L):
                t = g * L + tloc
                acc = jnp.zeros((L,), jnp.float32)
                for k in range(C // L):
                    a = es_v[slot, t, pl.ds(k * L, L)]
                    b = wr_v[slot, t, pl.ds(k * L, L)]
                    d = eo_v[slot, t, pl.ds(k * L, L)]
                    acc = acc + a * b * d
                sums = jnp.where(lanes == tloc, jnp.sum(acc), sums)
            out_v[pl.ds(j * CH + g * L, L)] = sums
            return carry2

        lax.fori_loop(0, CH // L, group_body, 0)

    fire_chunk(0, 0, sems[0])
    for j in range(NCHUNK):
        nxt = j + 1
        if nxt < NCHUNK:
            fire_chunk(nxt, nxt % 2, sems[nxt % 2])
        drain_chunk(j % 2, sems[j % 2])
        compute_chunk(j, j % 2)

    pltpu.sync_copy(out_v, out_hbm.at[pl.ds(base, TPW)])


def kernel(initializations, rel_weights, sro_triples):
    s = sro_triples[0].reshape(NW, NCHUNK, CH)
    r = sro_triples[1].reshape(NW, NCHUNK, CH)
    o = sro_triples[2].reshape(NW, NCHUNK, CH)
    return _distmult_sc(initializations, rel_weights, s, r, o)
